# baseline (speedup 1.0000x reference)
import functools

import jax
import jax.numpy as jnp
from jax import lax
from jax.experimental import pallas as pl

_B, _T, _D, _K = 4, 1024, 32, 8192
_N = _B * _T
_NUM_NEG = 16
_TEMP = 0.1
_MARGIN = 0.2
_FEAT_W = 1.0
_TRIP_W = 1.0
_CONTR_W = 0.5
_BN = 256
_NB = _N // _BN
_EPS = 1e-12
_QBITS = 13
_QMASK = (1 << _QBITS) - 1
_QSCALE = 4095.0
_NSPLIT = 16
_KQ = _K // _NSPLIT


def _tree_min(x):
    r = x.shape[0]
    while r > 8:
        r //= 2
        x = jnp.minimum(x[:r], x[r:])
    return jnp.min(x, axis=0, keepdims=True)


def _tree_max(x):
    r = x.shape[0]
    while r > 8:
        r //= 2
        x = jnp.maximum(x[:r], x[r:])
    return jnp.max(x, axis=0, keepdims=True)


def _extract16(kq, imax):
    gms = []
    gm = _tree_min(kq)
    gms.append(gm)
    for _ in range(_NUM_NEG - 1):
        shift = imax - gm
        v = kq + shift
        gm = _tree_min(v) - shift
        gms.append(gm)
    return gms


def _block_body(s_ref, t_ref, nt_ref, code_ref, cb_ref, out_ref):
    s = s_ref[...]
    t = t_ref[...]
    nt = nt_ref[...]
    code = code_ref[...].reshape(1, _BN)
    cb = cb_ref[...]

    diff = s - t
    sq = jnp.sum(diff * diff, axis=1, keepdims=True)
    feat = sq / _D
    pos_dist = jnp.sqrt(sq)
    dn = s - nt
    neg_dist = jnp.sqrt(jnp.sum(dn * dn, axis=1, keepdims=True))
    trip = jnp.maximum(pos_dist - neg_dist + _MARGIN, 0.0)

    c2col = jnp.sum(cb * cb, axis=1, keepdims=True)
    rc_col = 1.0 / jnp.maximum(jnp.sqrt(c2col), _EPS)
    x2 = jnp.sum(s * s, axis=1, keepdims=True)
    rx = 1.0 / jnp.maximum(jnp.sqrt(x2), _EPS)

    ones_bn = jnp.ones((_BN, 1), jnp.float32)
    s_aug = jnp.concatenate([-2.0 * s, ones_bn, x2], axis=1)
    ones_k = jnp.ones((_K, 1), jnp.float32)
    cb_aug = jnp.concatenate([cb, c2col + 2.0, ones_k], axis=1)
    d2 = lax.dot_general(cb_aug, s_aug, (((1,), (1,)), ((), ())),
                         preferred_element_type=jnp.float32)
    sn_aug = jnp.concatenate([s * rx, ones_bn], axis=1)
    cbn_aug = jnp.concatenate([cb * (rc_col * _QSCALE),
                               jnp.full((_K, 1), _QSCALE, jnp.float32)],
                              axis=1)
    cosqf = lax.dot_general(cbn_aug, sn_aug, (((1,), (1,)), ((), ())),
                            preferred_element_type=jnp.float32)

    row = lax.broadcasted_iota(jnp.int32, (_K, _BN), 0)
    is_pos = row == code
    d2 = jnp.where(is_pos, jnp.inf, d2)
    cosq = jnp.maximum(cosqf, 0.0).astype(jnp.int32)
    kq = jnp.bitwise_or(
        jnp.bitwise_and(lax.bitcast_convert_type(d2, jnp.int32),
                        jnp.int32(~_QMASK)),
        cosq)

    posk = _tree_max(kq)

    imax = jnp.int32(jnp.iinfo(jnp.int32).max)
    cand = []
    for q in range(_NSPLIT):
        cand += _extract16(kq[q * _KQ:(q + 1) * _KQ, :], imax)
    cand64 = jnp.concatenate(cand, axis=0)
    gms = _extract16(cand64, imax)

    inv_q = jnp.float32(1.0 / (_QSCALE * _TEMP))
    off = jnp.float32(-1.0 / _TEMP)
    qmask = jnp.int32(_QMASK)
    logits = [jnp.bitwise_and(posk, qmask).astype(jnp.float32) * inv_q + off]
    for gm_i in gms:
        logits.append(
            jnp.bitwise_and(gm_i, qmask).astype(jnp.float32) * inv_q + off)
    mmax = functools.reduce(jnp.maximum, logits)
    sumexp = sum(jnp.exp(l - mmax) for l in logits)
    ce = mmax + jnp.log(sumexp) - logits[0]

    feat_part = jnp.sum(feat)
    trip_part = jnp.sum(trip)
    ce_part = jnp.sum(ce)

    r = lax.broadcasted_iota(jnp.int32, (1, 8, 128), 1)
    c = lax.broadcasted_iota(jnp.int32, (1, 8, 128), 2)
    partial = (jnp.where((r == 0) & (c == 0), feat_part, 0.0)
               + jnp.where((r == 0) & (c == 1), trip_part, 0.0)
               + jnp.where((r == 0) & (c == 2), ce_part, 0.0))
    out_ref[...] = partial


def _fused_loss(student_flat, teacher_flat, negteacher_flat, codes3d, codebook):
    out = pl.pallas_call(
        _block_body,
        grid=(_NB,),
        in_specs=[
            pl.BlockSpec((_BN, _D), lambda i: (i, 0)),
            pl.BlockSpec((_BN, _D), lambda i: (i, 0)),
            pl.BlockSpec((_BN, _D), lambda i: (i, 0)),
            pl.BlockSpec((1, 1, _BN), lambda i: (i, 0, 0)),
            pl.BlockSpec((_K, _D), lambda i: (0, 0)),
        ],
        out_specs=pl.BlockSpec((1, 8, 128), lambda i: (i, 0, 0)),
        out_shape=jax.ShapeDtypeStruct((_NB, 8, 128), jnp.float32),
    )(student_flat, teacher_flat, negteacher_flat, codes3d, codebook)
    return out


def kernel(student_features, teacher_features, teacher_codes, codebook):
    student_flat = student_features.reshape(_N, _D)
    teacher_flat = teacher_features.reshape(_N, _D)
    negteacher_flat = jnp.roll(teacher_features, shift=1, axis=0).reshape(_N, _D)
    codes3d = teacher_codes.reshape(_NB, 1, _BN).astype(jnp.int32)

    parts = _fused_loss(student_flat, teacher_flat, negteacher_flat,
                        codes3d, codebook)
    sums = jnp.sum(parts, axis=0)
    feat_sum = sums[0, 0]
    trip_sum = sums[0, 1]
    ce_sum = sums[0, 2]
    inv_n = 1.0 / _N
    return (_FEAT_W * feat_sum * inv_n
            + _TRIP_W * trip_sum * inv_n
            + _CONTR_W * ce_sum * inv_n)


# speedup vs baseline: 1.0734x; 1.0734x over previous
import functools

import jax
import jax.numpy as jnp
from jax import lax
from jax.experimental import pallas as pl

_B, _T, _D, _K = 4, 1024, 32, 8192
_N = _B * _T
_NUM_NEG = 16
_TEMP = 0.1
_MARGIN = 0.2
_FEAT_W = 1.0
_TRIP_W = 1.0
_CONTR_W = 0.5
_BN = 256
_NB = _N // _BN
_EPS = 1e-12
_QBITS = 13
_QMASK = (1 << _QBITS) - 1
_QSCALE = 4095.0
_NSPLIT = 16
_KQ = _K // _NSPLIT


def _tree_min(x):
    r = x.shape[0]
    while r > 8:
        r //= 2
        x = jnp.minimum(x[:r], x[r:])
    return jnp.min(x, axis=0, keepdims=True)


def _tree_max(x):
    r = x.shape[0]
    while r > 8:
        r //= 2
        x = jnp.maximum(x[:r], x[r:])
    return jnp.max(x, axis=0, keepdims=True)


def _extract16(kq, imax):
    gms = []
    gm = jnp.min(kq, axis=0, keepdims=True)
    gms.append(gm)
    for _ in range(_NUM_NEG - 1):
        shift = imax - gm
        v = kq + shift
        gm = jnp.min(v, axis=0, keepdims=True) - shift
        gms.append(gm)
    return gms


def _block_body(s_ref, t_ref, nt_ref, code_ref, cb_ref, out_ref):
    s = s_ref[...]
    t = t_ref[...]
    nt = nt_ref[...]
    code = code_ref[...].reshape(1, _BN)
    cb = cb_ref[...]

    diff = s - t
    sq = jnp.sum(diff * diff, axis=1, keepdims=True)
    feat = sq / _D
    pos_dist = jnp.sqrt(sq)
    dn = s - nt
    neg_dist = jnp.sqrt(jnp.sum(dn * dn, axis=1, keepdims=True))
    trip = jnp.maximum(pos_dist - neg_dist + _MARGIN, 0.0)

    c2col = jnp.sum(cb * cb, axis=1, keepdims=True)
    rc_col = 1.0 / jnp.maximum(jnp.sqrt(c2col), _EPS)
    x2 = jnp.sum(s * s, axis=1, keepdims=True)
    rx = 1.0 / jnp.maximum(jnp.sqrt(x2), _EPS)

    ones_bn = jnp.ones((_BN, 1), jnp.float32)
    s_aug = jnp.concatenate([-2.0 * s, ones_bn, x2], axis=1)
    ones_k = jnp.ones((_K, 1), jnp.float32)
    cb_aug = jnp.concatenate([cb, c2col + 2.0, ones_k], axis=1)
    d2 = lax.dot_general(cb_aug, s_aug, (((1,), (1,)), ((), ())),
                         preferred_element_type=jnp.float32)
    sn_aug = jnp.concatenate([s * rx, ones_bn], axis=1)
    cbn_aug = jnp.concatenate([cb * (rc_col * _QSCALE),
                               jnp.full((_K, 1), _QSCALE, jnp.float32)],
                              axis=1)
    cosqf = lax.dot_general(cbn_aug, sn_aug, (((1,), (1,)), ((), ())),
                            preferred_element_type=jnp.float32)

    row = lax.broadcasted_iota(jnp.int32, (_K, _BN), 0)
    is_pos = row == code
    d2 = jnp.where(is_pos, jnp.inf, d2)
    cosq = jnp.maximum(cosqf, 0.0).astype(jnp.int32)
    kq = jnp.bitwise_or(
        jnp.bitwise_and(lax.bitcast_convert_type(d2, jnp.int32),
                        jnp.int32(~_QMASK)),
        cosq)

    posk = jnp.max(kq, axis=0, keepdims=True)

    imax = jnp.int32(jnp.iinfo(jnp.int32).max)
    cand = []
    for q in range(_NSPLIT):
        cand += _extract16(kq[q * _KQ:(q + 1) * _KQ, :], imax)
    cand64 = jnp.concatenate(cand, axis=0)
    gms = _extract16(cand64, imax)

    inv_q = jnp.float32(1.0 / (_QSCALE * _TEMP))
    off = jnp.float32(-1.0 / _TEMP)
    qmask = jnp.int32(_QMASK)
    logits = [jnp.bitwise_and(posk, qmask).astype(jnp.float32) * inv_q + off]
    for gm_i in gms:
        logits.append(
            jnp.bitwise_and(gm_i, qmask).astype(jnp.float32) * inv_q + off)
    mmax = functools.reduce(jnp.maximum, logits)
    sumexp = sum(jnp.exp(l - mmax) for l in logits)
    ce = mmax + jnp.log(sumexp) - logits[0]

    feat_part = jnp.sum(feat)
    trip_part = jnp.sum(trip)
    ce_part = jnp.sum(ce)

    r = lax.broadcasted_iota(jnp.int32, (8, 128), 0)
    c = lax.broadcasted_iota(jnp.int32, (8, 128), 1)
    partial = (jnp.where((r == 0) & (c == 0), feat_part, 0.0)
               + jnp.where((r == 0) & (c == 1), trip_part, 0.0)
               + jnp.where((r == 0) & (c == 2), ce_part, 0.0))

    @pl.when(pl.program_id(0) == 0)
    def _init():
        out_ref[...] = partial

    @pl.when(pl.program_id(0) != 0)
    def _acc():
        out_ref[...] += partial


_BLK_PER_B = _T // _BN
_ROLL_OFF = (_B - 1) * _BLK_PER_B


def _fused_loss(student_flat, teacher_flat, codes3d, codebook):
    out = pl.pallas_call(
        _block_body,
        grid=(_NB,),
        in_specs=[
            pl.BlockSpec((_BN, _D), lambda i: (i, 0)),
            pl.BlockSpec((_BN, _D), lambda i: (i, 0)),
            pl.BlockSpec((_BN, _D),
                         lambda i: (lax.rem(i + _ROLL_OFF, _NB), 0)),
            pl.BlockSpec((1, 1, _BN), lambda i: (i, 0, 0)),
            pl.BlockSpec((_K, _D), lambda i: (0, 0)),
        ],
        out_specs=pl.BlockSpec((8, 128), lambda i: (0, 0)),
        out_shape=jax.ShapeDtypeStruct((8, 128), jnp.float32),
    )(student_flat, teacher_flat, teacher_flat, codes3d, codebook)
    return out


def kernel(student_features, teacher_features, teacher_codes, codebook):
    student_flat = student_features.reshape(_N, _D)
    teacher_flat = teacher_features.reshape(_N, _D)
    codes3d = teacher_codes.reshape(_NB, 1, _BN).astype(jnp.int32)

    sums = _fused_loss(student_flat, teacher_flat, codes3d, codebook)
    feat_sum = sums[0, 0]
    trip_sum = sums[0, 1]
    ce_sum = sums[0, 2]
    inv_n = 1.0 / _N
    return (_FEAT_W * feat_sum * inv_n
            + _TRIP_W * trip_sum * inv_n
            + _CONTR_W * ce_sum * inv_n)
